# Initial kernel scaffold; baseline (speedup 1.0000x reference)
import jax
import jax.numpy as jnp
from jax.experimental import pallas as pl


def kernel(x, gate_w, w1, w2, w3):
    raise NotImplementedError("write your pallas kernel here")


# speedup vs baseline: 2.1621x; 2.1621x over previous
import jax
import jax.numpy as jnp
from jax.experimental import pallas as pl
from jax.experimental.pallas import tpu as pltpu

E = 64
TOP_K = 2
DIM = 768
HID = 2048
T = 32
P = T * TOP_K


def _gate_kernel(x_ref, gw_ref, sidx_ref, swt_ref):
    xf = x_ref[...]
    gw = gw_ref[...]
    lt = jax.lax.dot_general(gw, xf, (((1,), (1,)), ((), ())),
                             preferred_element_type=jnp.float32)
    m = jnp.max(lt, axis=0, keepdims=True)
    p = jnp.exp(lt - m)
    prob = p / jnp.sum(p, axis=0, keepdims=True)

    rows = jax.lax.broadcasted_iota(jnp.int32, (E, T), 0)
    m1 = jnp.max(prob, axis=0, keepdims=True)
    i1 = jnp.min(jnp.where(prob == m1, rows, E), axis=0, keepdims=True)
    pm = jnp.where(rows == i1, -1.0, prob)
    m2 = jnp.max(pm, axis=0, keepdims=True)
    i2 = jnp.min(jnp.where(pm == m2, rows, E), axis=0, keepdims=True)
    s = m1 + m2 + 1e-20
    w1n = m1 / s
    w2n = m2 / s

    e_vec = jnp.concatenate([i1, i2], axis=1)
    w_vec = jnp.concatenate([w1n, w2n], axis=1)
    cols = jax.lax.broadcasted_iota(jnp.int32, (1, P), 1)
    t_vec = cols % T

    key0 = e_vec * P + cols
    big = jnp.int32(E * P + P)

    def body(i, carry):
        key, se, st, sw = carry
        mk = jnp.min(key, axis=1, keepdims=True)
        sel = key == mk
        e_i = jnp.sum(jnp.where(sel, e_vec, 0), axis=1, keepdims=True)
        t_i = jnp.sum(jnp.where(sel, t_vec, 0), axis=1, keepdims=True)
        w_i = jnp.sum(jnp.where(sel, w_vec, 0.0), axis=1, keepdims=True)
        at = cols == i
        se = jnp.where(at, e_i, se)
        st = jnp.where(at, t_i, st)
        sw = jnp.where(at, w_i, sw)
        key = jnp.where(sel, big, key)
        return key, se, st, sw

    init = (key0, jnp.zeros_like(e_vec), jnp.zeros_like(e_vec),
            jnp.zeros_like(w_vec))
    _, se, st, sw = jax.lax.fori_loop(0, P, body, init)

    sidx_ref[0:1, :] = se
    sidx_ref[1:2, :] = st
    swt_ref[...] = sw


def _ffn_kernel(sidx_ref, swt_ref, x_ref, w1_ref, w3_ref, w2_ref, out_ref):
    q = pl.program_id(0)

    @pl.when(q == 0)
    def _init():
        out_ref[...] = jnp.zeros_like(out_ref)

    t = sidx_ref[1, q]
    row = x_ref[pl.ds(t, 1), :]
    a = jax.lax.dot_general(row, w1_ref[0], (((1,), (1,)), ((), ())),
                            preferred_element_type=jnp.float32)
    b = jax.lax.dot_general(row, w3_ref[0], (((1,), (1,)), ((), ())),
                            preferred_element_type=jnp.float32)
    h = a * jax.nn.sigmoid(a) * b
    o = jax.lax.dot_general(h, w2_ref[0], (((1,), (1,)), ((), ())),
                            preferred_element_type=jnp.float32)
    w = swt_ref[0, q]
    out_ref[pl.ds(t, 1), :] = out_ref[pl.ds(t, 1), :] + o * w


def kernel(x, gate_w, w1, w2, w3):
    orig_shape = x.shape
    xf = x.reshape(-1, DIM)

    sidx, swt = pl.pallas_call(
        _gate_kernel,
        out_shape=(
            jax.ShapeDtypeStruct((2, P), jnp.int32),
            jax.ShapeDtypeStruct((1, P), jnp.float32),
        ),
    )(xf, gate_w)

    grid_spec = pltpu.PrefetchScalarGridSpec(
        num_scalar_prefetch=2,
        grid=(P,),
        in_specs=[
            pl.BlockSpec((T, DIM), lambda q, sidx, swt: (0, 0)),
            pl.BlockSpec((1, HID, DIM), lambda q, sidx, swt: (sidx[0, q], 0, 0)),
            pl.BlockSpec((1, HID, DIM), lambda q, sidx, swt: (sidx[0, q], 0, 0)),
            pl.BlockSpec((1, DIM, HID), lambda q, sidx, swt: (sidx[0, q], 0, 0)),
        ],
        out_specs=pl.BlockSpec((T, DIM), lambda q, sidx, swt: (0, 0)),
    )

    out = pl.pallas_call(
        _ffn_kernel,
        grid_spec=grid_spec,
        out_shape=jax.ShapeDtypeStruct((T, DIM), jnp.float32),
        compiler_params=pltpu.CompilerParams(
            dimension_semantics=("arbitrary",),
        ),
    )(sidx, swt, xf, w1, w3, w2)

    return out.reshape(orig_shape)
